# baseline (speedup 1.0000x reference)
import jax
import jax.numpy as jnp
from jax.experimental import pallas as pl
from jax.experimental.pallas import tpu as pltpu

_NS = 32.0
_CB = 8


def _pool_kernel(cx_ref, cy_ref, px_ref, py_ref, hid_ref, w3_ref, b_ref,
                 out_ref):
    npd, hidd = hid_ref.shape
    ncell = w3_ref.shape[0]
    g = 8
    cell_sz = _NS / g
    prog = pl.program_id(0)

    cx = cx_ref[0]
    cy = cy_ref[0]
    relx = px_ref[:] - cx
    rely = py_ref[:] - cy
    inb = (jnp.abs(relx) <= _NS / 2) & (jnp.abs(rely) <= _NS / 2)
    gx = jnp.clip(jnp.floor((relx + _NS / 2) / cell_sz).astype(jnp.int32),
                  0, g - 1)
    gy = jnp.clip(jnp.floor((rely + _NS / 2) / cell_sz).astype(jnp.int32),
                  0, g - 1)
    jid = jax.lax.broadcasted_iota(jnp.int32, (npd, _CB), 0)
    cid = prog * _CB + jax.lax.broadcasted_iota(jnp.int32, (npd, _CB), 1)
    valid = inb & (jid != cid)
    cellv = jnp.where(valid, gy * g + gx, -1)

    hr = jnp.maximum(hid_ref[:], 0.0)

    def body(c, acc):
        rows = []
        for k in range(_CB):
            m = cellv[:, k:k + 1] == c
            rows.append(jnp.max(jnp.where(m, hr, 0.0), axis=0)[None, :])
        reds = jnp.concatenate(rows, axis=0)
        wc = w3_ref[c]
        return acc + jnp.dot(reds, wc, preferred_element_type=jnp.float32)

    acc = jax.lax.fori_loop(0, ncell, body,
                            jnp.zeros((_CB, hidd), jnp.float32))
    out_ref[:] = acc + b_ref[:]


def kernel(hidden_states, positions, W, b):
    npd, hidd = hidden_states.shape
    fan_in = W.shape[1]
    ncell = fan_in // hidd
    nprog = npd // _CB

    px = positions[:, 0:1]
    py = positions[:, 1:2]
    cx = positions[:, 0].reshape(nprog, 1, _CB)
    cy = positions[:, 1].reshape(nprog, 1, _CB)
    w3 = W.reshape(hidd, ncell, hidd).transpose(1, 2, 0)
    b2 = b.reshape(1, hidd)

    out = pl.pallas_call(
        _pool_kernel,
        grid=(nprog,),
        in_specs=[
            pl.BlockSpec((1, 1, _CB), lambda i: (i, 0, 0)),
            pl.BlockSpec((1, 1, _CB), lambda i: (i, 0, 0)),
            pl.BlockSpec((npd, 1), lambda i: (0, 0)),
            pl.BlockSpec((npd, 1), lambda i: (0, 0)),
            pl.BlockSpec((npd, hidd), lambda i: (0, 0)),
            pl.BlockSpec((ncell, hidd, hidd), lambda i: (0, 0, 0)),
            pl.BlockSpec((1, hidd), lambda i: (0, 0)),
        ],
        out_specs=pl.BlockSpec((_CB, hidd), lambda i: (i, 0)),
        out_shape=jax.ShapeDtypeStruct((npd, hidd), jnp.float32),
    )(cx, cy, px, py, hidden_states, w3, b2)
    return out


# speedup vs baseline: 8.2086x; 8.2086x over previous
import functools

import jax
import jax.numpy as jnp
from jax import lax
from jax.experimental import pallas as pl
from jax.experimental.pallas import tpu as pltpu
from jax.experimental.pallas import tpu_sc as plsc

_NS = 32.0
_G = 8
_NCELL = _G * _G
_HID = 128
_NP = 1024
_L = 16

_NWORK = 32
_CPW = _NP // _NWORK
_GRP = 8
_NGRP = _CPW // _GRP
_CHUNK = 256
_NCHUNK = _NP // _CHUNK
_DUMMY = _GRP * _NCELL
_ROWS = _GRP * _NCELL + 1


def _lane_splat(v, lane):
    idx = jnp.full((_L,), lane, dtype=jnp.int32)
    dn = lax.GatherDimensionNumbers(
        offset_dims=(), collapsed_slice_dims=(0,), start_index_map=(0,))
    return lax.gather(v, idx[:, None], dn, (1,),
                      mode=lax.GatherScatterMode.PROMISE_IN_BOUNDS)


def _sc_pool(hid_hbm, px_hbm, py_hbm, out_hbm, pxv, pyv, cellr, grids, hbuf):
    wid = lax.axis_index("s") * 2 + lax.axis_index("c")
    c0 = wid * _CPW
    pltpu.sync_copy(px_hbm, pxv)
    pltpu.sync_copy(py_hbm, pyv)
    iota = lax.iota(jnp.int32, _L)
    half = _NS / 2
    inv = _G / _NS
    zero16 = jnp.zeros((_L,), jnp.float32)

    def group_body(g, carry):
        gbase = c0 + g * _GRP
        cxv = pxv[pl.ds(gbase, _L)]
        cyv = pyv[pl.ds(gbase, _L)]

        def zb(i, c):
            grids[pl.ds(i * _L, _L)] = zero16
            return c
        lax.fori_loop(0, _ROWS * _HID // _L, zb, 0)

        for k in range(_GRP):
            cxk = _lane_splat(cxv, k)
            cyk = _lane_splat(cyv, k)
            cidk = gbase + k

            def cb(jg, c):
                jb = jg * _L
                jvec = jb + iota
                rx = pxv[pl.ds(jb, _L)] - cxk
                ry = pyv[pl.ds(jb, _L)] - cyk
                inb = (jnp.abs(rx) <= half) & (jnp.abs(ry) <= half)
                gx = jnp.clip((rx + half) * inv, 0.0, _G - 1.0)
                gy = jnp.clip((ry + half) * inv, 0.0, _G - 1.0)
                cell = gy.astype(jnp.int32) * _G + gx.astype(jnp.int32)
                ns = jvec != jnp.full((_L,), cidk, jnp.int32)
                row = jnp.where(inb & ns, cell + k * _NCELL, _DUMMY) * _HID
                plsc.store_scatter(cellr, [jvec * _L + k], row)
                return c
            lax.fori_loop(0, _NP // _L, cb, 0)

        def ch_body(c, cc):
            pltpu.sync_copy(hid_hbm.at[pl.ds(c * _CHUNK, _CHUNK), :], hbuf)

            def j_body(j, jc):
                jj = c * _CHUNK + j
                rowv = cellr[pl.ds(jj * _L, _L)]
                hv = [hbuf[j, pl.ds(t * _L, _L)] for t in range(_HID // _L)]
                for k in range(_GRP):
                    rk = _lane_splat(rowv, k)
                    for t in range(_HID // _L):
                        idx = rk + (t * _L + iota)
                        gv = plsc.load_gather(grids, [idx])
                        plsc.store_scatter(grids, [idx],
                                           jnp.maximum(gv, hv[t]))
                return jc
            lax.fori_loop(0, _CHUNK, j_body, 0)
            return cc
        lax.fori_loop(0, _NCHUNK, ch_body, 0)

        pltpu.sync_copy(grids.at[pl.ds(0, _GRP * _NCELL * _HID)],
                        out_hbm.at[pl.ds(gbase * _NCELL * _HID,
                                         _GRP * _NCELL * _HID)])
        return carry
    lax.fori_loop(0, _NGRP, group_body, 0)


def _mm_kernel(f_ref, wt_ref, b_ref, o_ref):
    o_ref[:] = (jnp.dot(f_ref[:], wt_ref[:],
                        preferred_element_type=jnp.float32) + b_ref[:])


def kernel(hidden_states, positions, W, b):
    npd, hidd = hidden_states.shape
    px = positions[:, 0]
    py = positions[:, 1]

    pool = pl.kernel(
        _sc_pool,
        out_type=jax.ShapeDtypeStruct((_NP * _NCELL * _HID,), jnp.float32),
        mesh=plsc.VectorSubcoreMesh(core_axis_name="c", subcore_axis_name="s"),
        compiler_params=pltpu.CompilerParams(needs_layout_passes=False),
        scratch_types=[
            pltpu.VMEM((_NP,), jnp.float32),
            pltpu.VMEM((_NP,), jnp.float32),
            pltpu.VMEM((_NP * _L,), jnp.int32),
            pltpu.VMEM((_ROWS * _HID,), jnp.float32),
            pltpu.VMEM((_CHUNK, _HID), jnp.float32),
        ],
    )
    flat = pool(hidden_states, px, py).reshape(npd, _NCELL * hidd)

    rb = 128
    out = pl.pallas_call(
        _mm_kernel,
        grid=(npd // rb,),
        in_specs=[
            pl.BlockSpec((rb, _NCELL * hidd), lambda i: (i, 0)),
            pl.BlockSpec((_NCELL * hidd, hidd), lambda i: (0, 0)),
            pl.BlockSpec((1, hidd), lambda i: (0, 0)),
        ],
        out_specs=pl.BlockSpec((rb, hidd), lambda i: (i, 0)),
        out_shape=jax.ShapeDtypeStruct((npd, hidd), jnp.float32),
    )(flat, W.T, b.reshape(1, hidd))
    return out


# speedup vs baseline: 8.6460x; 1.0533x over previous
import functools

import jax
import jax.numpy as jnp
from jax import lax
from jax.experimental import pallas as pl
from jax.experimental.pallas import tpu as pltpu
from jax.experimental.pallas import tpu_sc as plsc

_NS = 32.0
_G = 8
_NCELL = _G * _G
_HID = 128
_NP = 1024
_L = 16

_NWORK = 32
_CPW = _NP // _NWORK
_GRP = 8
_NGRP = _CPW // _GRP
_CHUNK = 256
_NCHUNK = _NP // _CHUNK
_DUMMY = _GRP * _NCELL
_ROWS = _GRP * _NCELL + 1


def _lane_splat(v, lane):
    idx = jnp.full((_L,), lane, dtype=jnp.int32)
    dn = lax.GatherDimensionNumbers(
        offset_dims=(), collapsed_slice_dims=(0,), start_index_map=(0,))
    return lax.gather(v, idx[:, None], dn, (1,),
                      mode=lax.GatherScatterMode.PROMISE_IN_BOUNDS)


def _sc_pool(hid_hbm, px_hbm, py_hbm, out_hbm, pxv, pyv, cellr, hbuf, *gbufs):
    wid = lax.axis_index("s") * 2 + lax.axis_index("c")
    c0 = wid * _CPW
    pltpu.sync_copy(px_hbm, pxv)
    pltpu.sync_copy(py_hbm, pyv)
    iota = lax.iota(jnp.int32, _L)
    half = _NS / 2
    inv = _G / _NS
    zero16 = jnp.zeros((_L,), jnp.float32)

    def group_body(g, carry):
        gbase = c0 + g * _GRP
        cxv = pxv[pl.ds(gbase, _L)]
        cyv = pyv[pl.ds(gbase, _L)]

        def zb(i, c):
            for k in range(_GRP):
                gbufs[k][pl.ds(i * _L, _L)] = zero16
            return c
        lax.fori_loop(0, (_NCELL + 1) * _HID // _L, zb, 0)

        for k in range(_GRP):
            cxk = _lane_splat(cxv, k)
            cyk = _lane_splat(cyv, k)
            cidk = gbase + k

            def cb(jg, c):
                jb = jg * _L
                jvec = jb + iota
                rx = pxv[pl.ds(jb, _L)] - cxk
                ry = pyv[pl.ds(jb, _L)] - cyk
                inb = (jnp.abs(rx) <= half) & (jnp.abs(ry) <= half)
                gx = jnp.clip((rx + half) * inv, 0.0, _G - 1.0)
                gy = jnp.clip((ry + half) * inv, 0.0, _G - 1.0)
                cell = gy.astype(jnp.int32) * _G + gx.astype(jnp.int32)
                ns = jvec != jnp.full((_L,), cidk, jnp.int32)
                row = jnp.where(inb & ns, cell, _NCELL) * _HID
                plsc.store_scatter(cellr, [jvec * _L + k], row)
                return c
            lax.fori_loop(0, _NP // _L, cb, 0)

        def ch_body(c, cc):
            pltpu.sync_copy(hid_hbm.at[pl.ds(c * _CHUNK, _CHUNK), :], hbuf)

            def j_body(j, jc):
                jj = c * _CHUNK + j
                rowv = cellr[pl.ds(jj * _L, _L)]
                hv = [hbuf[j, pl.ds(t * _L, _L)] for t in range(_HID // _L)]
                rks = [_lane_splat(rowv, k) for k in range(_GRP)]
                idxs = [[rks[k] + (t * _L + iota) for t in range(_HID // _L)]
                        for k in range(_GRP)]
                for t in range(_HID // _L):
                    for k in range(_GRP):
                        gv = plsc.load_gather(gbufs[k], [idxs[k][t]])
                        plsc.store_scatter(gbufs[k], [idxs[k][t]],
                                           jnp.maximum(gv, hv[t]))
                return jc
            lax.fori_loop(0, _CHUNK, j_body, 0)
            return cc
        lax.fori_loop(0, _NCHUNK, ch_body, 0)

        for k in range(_GRP):
            pltpu.sync_copy(
                gbufs[k].at[pl.ds(0, _NCELL * _HID)],
                out_hbm.at[pl.ds((gbase + k) * _NCELL * _HID, _NCELL * _HID)])
        return carry
    lax.fori_loop(0, _NGRP, group_body, 0)


def _mm_kernel(f_ref, wt_ref, b_ref, o_ref):
    o_ref[:] = (jnp.dot(f_ref[:], wt_ref[:],
                        preferred_element_type=jnp.float32) + b_ref[:])


def kernel(hidden_states, positions, W, b):
    npd, hidd = hidden_states.shape
    px = positions[:, 0]
    py = positions[:, 1]

    pool = pl.kernel(
        _sc_pool,
        out_type=jax.ShapeDtypeStruct((_NP * _NCELL * _HID,), jnp.float32),
        mesh=plsc.VectorSubcoreMesh(core_axis_name="c", subcore_axis_name="s"),
        compiler_params=pltpu.CompilerParams(needs_layout_passes=False),
        scratch_types=[
            pltpu.VMEM((_NP,), jnp.float32),
            pltpu.VMEM((_NP,), jnp.float32),
            pltpu.VMEM((_NP * _L,), jnp.int32),
            pltpu.VMEM((_CHUNK, _HID), jnp.float32),
        ] + [pltpu.VMEM(((_NCELL + 1) * _HID,), jnp.float32)
             for _ in range(_GRP)],
    )
    flat = pool(hidden_states, px, py).reshape(npd, _NCELL * hidd)

    rb = 128
    out = pl.pallas_call(
        _mm_kernel,
        grid=(npd // rb,),
        in_specs=[
            pl.BlockSpec((rb, _NCELL * hidd), lambda i: (i, 0)),
            pl.BlockSpec((_NCELL * hidd, hidd), lambda i: (0, 0)),
            pl.BlockSpec((1, hidd), lambda i: (0, 0)),
        ],
        out_specs=pl.BlockSpec((rb, hidd), lambda i: (i, 0)),
        out_shape=jax.ShapeDtypeStruct((npd, hidd), jnp.float32),
    )(flat, W.T, b.reshape(1, hidd))
    return out


# speedup vs baseline: 21.7715x; 2.5181x over previous
import functools

import jax
import jax.numpy as jnp
from jax import lax
from jax.experimental import pallas as pl
from jax.experimental.pallas import tpu as pltpu
from jax.experimental.pallas import tpu_sc as plsc

_NS = 32.0
_G = 8
_NCELL = _G * _G
_HID = 128
_NP = 1024
_L = 16

_NWORK = 32
_CPW = _NP // _NWORK
_GRP = 8
_NGRP = _CPW // _GRP
_CHUNK = 256
_NCHUNK = _NP // _CHUNK
_DUMMY = _GRP * _NCELL
_ROWS = _GRP * _NCELL + 1


def _lane_splat(v, lane):
    idx = jnp.full((_L,), lane, dtype=jnp.int32)
    dn = lax.GatherDimensionNumbers(
        offset_dims=(), collapsed_slice_dims=(0,), start_index_map=(0,))
    return lax.gather(v, idx[:, None], dn, (1,),
                      mode=lax.GatherScatterMode.PROMISE_IN_BOUNDS)


def _sc_pool(hid_hbm, px_hbm, py_hbm, out_hbm, pxv, pyv, cellr, hbuf, *gbufs):
    wid = lax.axis_index("s") * 2 + lax.axis_index("c")
    c0 = wid * _CPW
    pltpu.sync_copy(px_hbm, pxv)
    pltpu.sync_copy(py_hbm, pyv)
    iota = lax.iota(jnp.int32, _L)
    half = _NS / 2
    inv = _G / _NS
    zero16 = jnp.zeros((_L,), jnp.float32)

    def group_body(g, carry):
        gbase = c0 + g * _GRP
        cxv = pxv[pl.ds(gbase, _L)]
        cyv = pyv[pl.ds(gbase, _L)]

        def zb(i, c):
            for k in range(_GRP):
                gbufs[k][pl.ds(i * _L, _L)] = zero16
            return c
        lax.fori_loop(0, (_NCELL + 1) * _HID // _L, zb, 0)

        for k in range(_GRP):
            cxk = _lane_splat(cxv, k)
            cyk = _lane_splat(cyv, k)
            cidk = gbase + k

            def cb(jg, c):
                jb = jg * _L
                jvec = jb + iota
                rx = pxv[pl.ds(jb, _L)] - cxk
                ry = pyv[pl.ds(jb, _L)] - cyk
                inb = (jnp.abs(rx) <= half) & (jnp.abs(ry) <= half)
                gx = jnp.clip((rx + half) * inv, 0.0, _G - 1.0)
                gy = jnp.clip((ry + half) * inv, 0.0, _G - 1.0)
                cell = gy.astype(jnp.int32) * _G + gx.astype(jnp.int32)
                ns = jvec != jnp.full((_L,), cidk, jnp.int32)
                row = jnp.where(inb & ns, cell, _NCELL) * _HID
                plsc.store_scatter(cellr, [jvec * _L + k], row)
                return c
            lax.fori_loop(0, _NP // _L, cb, 0)

        def ch_body(c, cc):
            pltpu.sync_copy(hid_hbm.at[pl.ds(c * _CHUNK, _CHUNK), :], hbuf)

            def j_body(j, jc):
                nt = _HID // _L
                jj = c * _CHUNK + j
                rowv = cellr[pl.ds(jj * _L, _L)]
                hv = [hbuf[j, pl.ds(t * _L, _L)] for t in range(nt)]
                rks = [_lane_splat(rowv, k) for k in range(_GRP)]
                cvecs = [t * _L + iota for t in range(nt)]
                gs = [None] * nt
                ms = [None] * nt
                for t in range(nt):
                    gs[t] = [plsc.load_gather(gbufs[k], [rks[k] + cvecs[t]])
                             for k in range(_GRP)]
                    if t >= 1:
                        ms[t - 1] = [jnp.maximum(gs[t - 1][k], hv[t - 1])
                                     for k in range(_GRP)]
                        gs[t - 1] = None
                    if t >= 2:
                        for k in range(_GRP):
                            plsc.store_scatter(gbufs[k],
                                               [rks[k] + cvecs[t - 2]],
                                               ms[t - 2][k])
                        ms[t - 2] = None
                ms[nt - 1] = [jnp.maximum(gs[nt - 1][k], hv[nt - 1])
                              for k in range(_GRP)]
                for tt in (nt - 2, nt - 1):
                    for k in range(_GRP):
                        plsc.store_scatter(gbufs[k], [rks[k] + cvecs[tt]],
                                           ms[tt][k])
                return jc
            lax.fori_loop(0, _CHUNK, j_body, 0)
            return cc
        lax.fori_loop(0, _NCHUNK, ch_body, 0)

        for k in range(_GRP):
            pltpu.sync_copy(
                gbufs[k].at[pl.ds(0, _NCELL * _HID)],
                out_hbm.at[pl.ds((gbase + k) * _NCELL * _HID, _NCELL * _HID)])
        return carry
    lax.fori_loop(0, _NGRP, group_body, 0)


def _mm_kernel(f_ref, wt_ref, b_ref, o_ref):
    o_ref[:] = (jnp.dot(f_ref[:], wt_ref[:],
                        preferred_element_type=jnp.float32) + b_ref[:])


def kernel(hidden_states, positions, W, b):
    npd, hidd = hidden_states.shape
    px = positions[:, 0]
    py = positions[:, 1]

    pool = pl.kernel(
        _sc_pool,
        out_type=jax.ShapeDtypeStruct((_NP * _NCELL * _HID,), jnp.float32),
        mesh=plsc.VectorSubcoreMesh(core_axis_name="c", subcore_axis_name="s"),
        compiler_params=pltpu.CompilerParams(needs_layout_passes=False),
        scratch_types=[
            pltpu.VMEM((_NP,), jnp.float32),
            pltpu.VMEM((_NP,), jnp.float32),
            pltpu.VMEM((_NP * _L,), jnp.int32),
            pltpu.VMEM((_CHUNK, _HID), jnp.float32),
        ] + [pltpu.VMEM(((_NCELL + 1) * _HID,), jnp.float32)
             for _ in range(_GRP)],
    )
    flat = pool(hidden_states, px, py).reshape(npd, _NCELL * hidd)

    rb = 128
    out = pl.pallas_call(
        _mm_kernel,
        grid=(npd // rb,),
        in_specs=[
            pl.BlockSpec((rb, _NCELL * hidd), lambda i: (i, 0)),
            pl.BlockSpec((_NCELL * hidd, hidd), lambda i: (0, 0)),
            pl.BlockSpec((1, hidd), lambda i: (0, 0)),
        ],
        out_specs=pl.BlockSpec((rb, hidd), lambda i: (i, 0)),
        out_shape=jax.ShapeDtypeStruct((npd, hidd), jnp.float32),
    )(flat, W.T, b.reshape(1, hidd))
    return out


# speedup vs baseline: 22.2657x; 1.0227x over previous
import jax
import jax.numpy as jnp
from jax import lax
from jax.experimental import pallas as pl
from jax.experimental.pallas import tpu as pltpu
from jax.experimental.pallas import tpu_sc as plsc

_NS = 32.0
_G = 8
_NCELL = _G * _G
_HID = 128
_NP = 1024
_L = 16

_NWORK = 32
_CPW = _NP // _NWORK
_GRP = 8
_NGRP = _CPW // _GRP
_CHUNK = 256
_NCHUNK = _NP // _CHUNK
_ROWW = _HID // 2
_NT = _HID // (2 * _L)


def _lane_splat(v, lane):
    idx = jnp.full((_L,), lane, dtype=jnp.int32)
    dn = lax.GatherDimensionNumbers(
        offset_dims=(), collapsed_slice_dims=(0,), start_index_map=(0,))
    return lax.gather(v, idx[:, None], dn, (1,),
                      mode=lax.GatherScatterMode.PROMISE_IN_BOUNDS)


def _sc_pool(hid_hbm, px_hbm, py_hbm, out_hbm, pxv, pyv, cellr, hbuf, *gbufs):
    wid = lax.axis_index("s") * 2 + lax.axis_index("c")
    c0 = wid * _CPW
    pltpu.sync_copy(px_hbm, pxv)
    pltpu.sync_copy(py_hbm, pyv)
    iota = lax.iota(jnp.int32, _L)
    half = _NS / 2
    inv = _G / _NS
    zero16 = jnp.zeros((_L,), jnp.float32)

    def group_body(g, carry):
        gbase = c0 + g * _GRP
        cxv = pxv[pl.ds(gbase, _L)]
        cyv = pyv[pl.ds(gbase, _L)]

        def zb(i, c):
            for k in range(_GRP):
                gbufs[k][pl.ds(i * _L, _L)] = zero16
            return c
        lax.fori_loop(0, (_NCELL + 1) * _ROWW // _L, zb, 0)

        for k in range(_GRP):
            cxk = _lane_splat(cxv, k)
            cyk = _lane_splat(cyv, k)
            cidk = gbase + k

            def cb(jg, c):
                jb = jg * _L
                jvec = jb + iota
                rx = pxv[pl.ds(jb, _L)] - cxk
                ry = pyv[pl.ds(jb, _L)] - cyk
                inb = (jnp.abs(rx) <= half) & (jnp.abs(ry) <= half)
                gx = jnp.clip((rx + half) * inv, 0.0, _G - 1.0)
                gy = jnp.clip((ry + half) * inv, 0.0, _G - 1.0)
                cell = gy.astype(jnp.int32) * _G + gx.astype(jnp.int32)
                ns = jvec != jnp.full((_L,), cidk, jnp.int32)
                row = jnp.where(inb & ns, cell, _NCELL) * _ROWW
                plsc.store_scatter(cellr, [jvec * _L + k], row)
                return c
            lax.fori_loop(0, _NP // _L, cb, 0)

        def ch_body(c, cc):
            pltpu.sync_copy(hid_hbm.at[pl.ds(c * _CHUNK * _ROWW,
                                             _CHUNK * _ROWW)], hbuf)

            def j_body(j, jc):
                jj = c * _CHUNK + j
                rowv = cellr[pl.ds(jj * _L, _L)]
                hv = [plsc.bitcast(hbuf[pl.ds(j * _ROWW + t * _L, _L)],
                                   jnp.bfloat16)
                      for t in range(_NT)]
                rks = [_lane_splat(rowv, k) for k in range(_GRP)]
                cvecs = [t * _L + iota for t in range(_NT)]
                gs = [None] * _NT
                ms = [None] * _NT
                for t in range(_NT):
                    gs[t] = [plsc.load_gather(gbufs[k], [rks[k] + cvecs[t]])
                             for k in range(_GRP)]
                    if t >= 1:
                        ms[t - 1] = [
                            plsc.bitcast(jnp.maximum(
                                plsc.bitcast(gs[t - 1][k], jnp.bfloat16),
                                hv[t - 1]), jnp.float32)
                            for k in range(_GRP)]
                        gs[t - 1] = None
                    if t >= 2:
                        for k in range(_GRP):
                            plsc.store_scatter(gbufs[k],
                                               [rks[k] + cvecs[t - 2]],
                                               ms[t - 2][k])
                        ms[t - 2] = None
                ms[_NT - 1] = [
                    plsc.bitcast(jnp.maximum(
                        plsc.bitcast(gs[_NT - 1][k], jnp.bfloat16),
                        hv[_NT - 1]), jnp.float32)
                    for k in range(_GRP)]
                for tt in (_NT - 2, _NT - 1):
                    for k in range(_GRP):
                        plsc.store_scatter(gbufs[k], [rks[k] + cvecs[tt]],
                                           ms[tt][k])
                return jc
            lax.fori_loop(0, _CHUNK, j_body, 0)
            return cc
        lax.fori_loop(0, _NCHUNK, ch_body, 0)

        for k in range(_GRP):
            pltpu.sync_copy(
                gbufs[k].at[pl.ds(0, _NCELL * _ROWW)],
                out_hbm.at[pl.ds((gbase + k) * _NCELL * _ROWW,
                                 _NCELL * _ROWW)])
        return carry
    lax.fori_loop(0, _NGRP, group_body, 0)


def _mm_kernel(f_ref, wt_ref, b_ref, o_ref):
    o_ref[:] = (jnp.dot(f_ref[:].astype(jnp.float32), wt_ref[:],
                        preferred_element_type=jnp.float32) + b_ref[:])


def kernel(hidden_states, positions, W, b):
    npd, hidd = hidden_states.shape
    px = positions[:, 0]
    py = positions[:, 1]
    hid_w = lax.bitcast_convert_type(
        hidden_states.astype(jnp.bfloat16).reshape(npd * hidd // 2, 2),
        jnp.float32)

    pool = pl.kernel(
        _sc_pool,
        out_type=jax.ShapeDtypeStruct((_NP * _NCELL * _ROWW,), jnp.float32),
        mesh=plsc.VectorSubcoreMesh(core_axis_name="c", subcore_axis_name="s"),
        compiler_params=pltpu.CompilerParams(needs_layout_passes=False),
        scratch_types=[
            pltpu.VMEM((_NP,), jnp.float32),
            pltpu.VMEM((_NP,), jnp.float32),
            pltpu.VMEM((_NP * _L,), jnp.int32),
            pltpu.VMEM((_CHUNK * _ROWW,), jnp.float32),
        ] + [pltpu.VMEM(((_NCELL + 1) * _ROWW,), jnp.float32)
             for _ in range(_GRP)],
    )
    packed = pool(hid_w, px, py).reshape(npd, _NCELL * _ROWW)
    flat = lax.bitcast_convert_type(packed, jnp.bfloat16)
    flat = flat.reshape(npd, _NCELL * hidd)

    rb = 128
    out = pl.pallas_call(
        _mm_kernel,
        grid=(npd // rb,),
        in_specs=[
            pl.BlockSpec((rb, _NCELL * hidd), lambda i: (i, 0)),
            pl.BlockSpec((_NCELL * hidd, hidd), lambda i: (0, 0)),
            pl.BlockSpec((1, hidd), lambda i: (0, 0)),
        ],
        out_specs=pl.BlockSpec((rb, hidd), lambda i: (i, 0)),
        out_shape=jax.ShapeDtypeStruct((npd, hidd), jnp.float32),
    )(flat, W.T, b.reshape(1, hidd))
    return out


# speedup vs baseline: 33.5190x; 1.5054x over previous
import jax
import jax.numpy as jnp
from jax import lax
from jax.experimental import pallas as pl
from jax.experimental.pallas import tpu as pltpu
from jax.experimental.pallas import tpu_sc as plsc

_NS = 32.0
_G = 8
_NCELL = _G * _G
_HID = 128
_NP = 1024
_L = 16

_NWORK = 32
_CPW = _NP // _NWORK
_GRP = 8
_NGRP = _CPW // _GRP
_CHUNK = 256
_NCHUNK = _NP // _CHUNK
_ROWW = _HID // 2
_NT = _HID // (2 * _L)


def _lane_splat(v, lane):
    idx = jnp.full((_L,), lane, dtype=jnp.int32)
    dn = lax.GatherDimensionNumbers(
        offset_dims=(), collapsed_slice_dims=(0,), start_index_map=(0,))
    return lax.gather(v, idx[:, None], dn, (1,),
                      mode=lax.GatherScatterMode.PROMISE_IN_BOUNDS)


def _sc_pool(hid_hbm, px_hbm, py_hbm, out_hbm, pxv, pyv, cellr, hbuf, *gbufs):
    wid = lax.axis_index("s") * 2 + lax.axis_index("c")
    c0 = wid * _CPW
    pltpu.sync_copy(px_hbm, pxv)
    pltpu.sync_copy(py_hbm, pyv)
    iota = lax.iota(jnp.int32, _L)
    half = _NS / 2
    inv = _G / _NS
    zero16 = jnp.zeros((_L,), jnp.float32)

    def group_body(g, carry):
        gbase = c0 + g * _GRP
        cxv = pxv[pl.ds(gbase, _L)]
        cyv = pyv[pl.ds(gbase, _L)]

        def zb(i, c):
            for k in range(_GRP):
                gbufs[k][pl.ds(i * _L, _L)] = zero16
            return c
        lax.fori_loop(0, (_NCELL + 1) * _ROWW // _L, zb, 0)

        for k in range(_GRP):
            cxk = _lane_splat(cxv, k)
            cyk = _lane_splat(cyv, k)
            cidk = gbase + k

            def cb(jg, c):
                jb = jg * _L
                jvec = jb + iota
                rx = pxv[pl.ds(jb, _L)] - cxk
                ry = pyv[pl.ds(jb, _L)] - cyk
                inb = (jnp.abs(rx) <= half) & (jnp.abs(ry) <= half)
                gx = jnp.clip((rx + half) * inv, 0.0, _G - 1.0)
                gy = jnp.clip((ry + half) * inv, 0.0, _G - 1.0)
                cell = gy.astype(jnp.int32) * _G + gx.astype(jnp.int32)
                ns = jvec != jnp.full((_L,), cidk, jnp.int32)
                row = jnp.where(inb & ns, cell, _NCELL) * _ROWW
                plsc.store_scatter(cellr, [jvec * _L + k], row)
                return c
            lax.fori_loop(0, _NP // _L, cb, 0)

        def ch_body(c, cc):
            pltpu.sync_copy(hid_hbm.at[pl.ds(c * _CHUNK * _ROWW,
                                             _CHUNK * _ROWW)], hbuf)

            def j_body(j, jc):
                jj = c * _CHUNK + j
                rowv = cellr[pl.ds(jj * _L, _L)]
                hv = [plsc.bitcast(hbuf[pl.ds(j * _ROWW + t * _L, _L)],
                                   jnp.bfloat16)
                      for t in range(_NT)]
                rks = [_lane_splat(rowv, k) for k in range(_GRP)]
                cvecs = [t * _L + iota for t in range(_NT)]
                gs = [None] * _NT
                ms = [None] * _NT
                for t in range(_NT):
                    gs[t] = [plsc.load_gather(gbufs[k], [rks[k] + cvecs[t]])
                             for k in range(_GRP)]
                    if t >= 1:
                        ms[t - 1] = [
                            plsc.bitcast(jnp.maximum(
                                plsc.bitcast(gs[t - 1][k], jnp.bfloat16),
                                hv[t - 1]), jnp.float32)
                            for k in range(_GRP)]
                        gs[t - 1] = None
                    if t >= 2:
                        for k in range(_GRP):
                            plsc.store_scatter(gbufs[k],
                                               [rks[k] + cvecs[t - 2]],
                                               ms[t - 2][k])
                        ms[t - 2] = None
                ms[_NT - 1] = [
                    plsc.bitcast(jnp.maximum(
                        plsc.bitcast(gs[_NT - 1][k], jnp.bfloat16),
                        hv[_NT - 1]), jnp.float32)
                    for k in range(_GRP)]
                for tt in (_NT - 2, _NT - 1):
                    for k in range(_GRP):
                        plsc.store_scatter(gbufs[k], [rks[k] + cvecs[tt]],
                                           ms[tt][k])
                return jc
            lax.fori_loop(0, _CHUNK, j_body, 0)
            return cc
        lax.fori_loop(0, _NCHUNK, ch_body, 0)

        for k in range(_GRP):
            pltpu.sync_copy(
                gbufs[k].at[pl.ds(0, _NCELL * _ROWW)],
                out_hbm.at[pl.ds((gbase + k) * _NCELL * _ROWW,
                                 _NCELL * _ROWW)])
        return carry
    lax.fori_loop(0, _NGRP, group_body, 0)


def _mm_kernel(p_ref, we_ref, wo_ref, b_ref, o_ref):
    pw = lax.bitcast_convert_type(p_ref[:], jnp.int32)
    lowf = lax.bitcast_convert_type(pw << 16, jnp.float32)
    highf = lax.bitcast_convert_type(pw & jnp.int32(-65536), jnp.float32)
    o_ref[:] = (jnp.dot(lowf, we_ref[:], preferred_element_type=jnp.float32)
                + jnp.dot(highf, wo_ref[:],
                          preferred_element_type=jnp.float32)
                + b_ref[:])


def kernel(hidden_states, positions, W, b):
    npd, hidd = hidden_states.shape
    px = positions[:, 0]
    py = positions[:, 1]
    hu = lax.bitcast_convert_type(
        hidden_states.astype(jnp.bfloat16).astype(jnp.float32), jnp.uint32)
    hword = (hu[:, 0::2] >> 16) | (hu[:, 1::2] & jnp.uint32(0xFFFF0000))
    hid_w = lax.bitcast_convert_type(hword, jnp.float32).reshape(-1)

    pool = pl.kernel(
        _sc_pool,
        out_type=jax.ShapeDtypeStruct((_NP * _NCELL * _ROWW,), jnp.float32),
        mesh=plsc.VectorSubcoreMesh(core_axis_name="c", subcore_axis_name="s"),
        compiler_params=pltpu.CompilerParams(needs_layout_passes=False),
        scratch_types=[
            pltpu.VMEM((_NP,), jnp.float32),
            pltpu.VMEM((_NP,), jnp.float32),
            pltpu.VMEM((_NP * _L,), jnp.int32),
            pltpu.VMEM((_CHUNK * _ROWW,), jnp.float32),
        ] + [pltpu.VMEM(((_NCELL + 1) * _ROWW,), jnp.float32)
             for _ in range(_GRP)],
    )
    packed = pool(hid_w, px, py).reshape(npd, _NCELL * _ROWW)
    we_t = W[:, 0::2].T
    wo_t = W[:, 1::2].T

    rb = 128
    nw = _NCELL * _ROWW
    out = pl.pallas_call(
        _mm_kernel,
        grid=(npd // rb,),
        in_specs=[
            pl.BlockSpec((rb, nw), lambda i: (i, 0)),
            pl.BlockSpec((nw, hidd), lambda i: (0, 0)),
            pl.BlockSpec((nw, hidd), lambda i: (0, 0)),
            pl.BlockSpec((1, hidd), lambda i: (0, 0)),
        ],
        out_specs=pl.BlockSpec((rb, hidd), lambda i: (i, 0)),
        out_shape=jax.ShapeDtypeStruct((npd, hidd), jnp.float32),
    )(packed, we_t, wo_t, b.reshape(1, hidd))
    return out


# speedup vs baseline: 37.1879x; 1.1095x over previous
import jax
import jax.numpy as jnp
from jax import lax
from jax.experimental import pallas as pl
from jax.experimental.pallas import tpu as pltpu
from jax.experimental.pallas import tpu_sc as plsc

_NS = 32.0
_G = 8
_NCELL = _G * _G
_HID = 128
_NP = 1024
_L = 16

_NWORK = 32
_CPW = _NP // _NWORK
_GRP = 8
_NGRP = _CPW // _GRP
_CHUNK = 256
_NCHUNK = _NP // _CHUNK
_ROWW = _HID // 2
_NT = _HID // (2 * _L)


def _lane_splat(v, lane):
    idx = jnp.full((_L,), lane, dtype=jnp.int32)
    dn = lax.GatherDimensionNumbers(
        offset_dims=(), collapsed_slice_dims=(0,), start_index_map=(0,))
    return lax.gather(v, idx[:, None], dn, (1,),
                      mode=lax.GatherScatterMode.PROMISE_IN_BOUNDS)


def _sc_pool(hid_hbm, px_hbm, py_hbm, out_hbm, pxv, pyv, cellr, hbuf, *gbufs):
    wid = lax.axis_index("s") * 2 + lax.axis_index("c")
    c0 = wid * _CPW
    pltpu.sync_copy(px_hbm, pxv)
    pltpu.sync_copy(py_hbm, pyv)
    pltpu.sync_copy(hid_hbm, hbuf)
    iota = lax.iota(jnp.int32, _L)
    half = _NS / 2
    inv = _G / _NS
    zero16 = jnp.zeros((_L,), jnp.float32)

    def group_body(g, carry):
        gbase = c0 + g * _GRP
        cxv = pxv[pl.ds(gbase, _L)]
        cyv = pyv[pl.ds(gbase, _L)]

        def zb(i, c):
            for k in range(_GRP):
                gbufs[k][pl.ds(i * _L, _L)] = zero16
            return c
        lax.fori_loop(0, (_NCELL + 1) * _ROWW // _L, zb, 0)

        for k in range(_GRP):
            cxk = _lane_splat(cxv, k)
            cyk = _lane_splat(cyv, k)
            cidk = gbase + k

            def cb(jg, c):
                jb = jg * _L
                jvec = jb + iota
                rx = pxv[pl.ds(jb, _L)] - cxk
                ry = pyv[pl.ds(jb, _L)] - cyk
                inb = (jnp.abs(rx) <= half) & (jnp.abs(ry) <= half)
                gx = jnp.clip((rx + half) * inv, 0.0, _G - 1.0)
                gy = jnp.clip((ry + half) * inv, 0.0, _G - 1.0)
                cell = gy.astype(jnp.int32) * _G + gx.astype(jnp.int32)
                ns = jvec != jnp.full((_L,), cidk, jnp.int32)
                row = jnp.where(inb & ns, cell, _NCELL) * _ROWW
                plsc.store_scatter(cellr, [jvec * _L + k], row)
                return c
            lax.fori_loop(0, _NP // _L, cb, 0)

        if True:
            def j_body(j, jc):
                rowv = cellr[pl.ds(j * _L, _L)]
                hv = [plsc.bitcast(hbuf[pl.ds(j * _ROWW + t * _L, _L)],
                                   jnp.bfloat16)
                      for t in range(_NT)]
                rks = [_lane_splat(rowv, k) for k in range(_GRP)]
                cvecs = [t * _L + iota for t in range(_NT)]
                gs = [None] * _NT
                ms = [None] * _NT
                for t in range(_NT):
                    gs[t] = [plsc.load_gather(gbufs[k], [rks[k] + cvecs[t]])
                             for k in range(_GRP)]
                    if t >= 1:
                        ms[t - 1] = [
                            plsc.bitcast(jnp.maximum(
                                plsc.bitcast(gs[t - 1][k], jnp.bfloat16),
                                hv[t - 1]), jnp.float32)
                            for k in range(_GRP)]
                        gs[t - 1] = None
                    if t >= 2:
                        for k in range(_GRP):
                            plsc.store_scatter(gbufs[k],
                                               [rks[k] + cvecs[t - 2]],
                                               ms[t - 2][k])
                        ms[t - 2] = None
                ms[_NT - 1] = [
                    plsc.bitcast(jnp.maximum(
                        plsc.bitcast(gs[_NT - 1][k], jnp.bfloat16),
                        hv[_NT - 1]), jnp.float32)
                    for k in range(_GRP)]
                for tt in (_NT - 2, _NT - 1):
                    for k in range(_GRP):
                        plsc.store_scatter(gbufs[k], [rks[k] + cvecs[tt]],
                                           ms[tt][k])
                return jc
            lax.fori_loop(0, _NP, j_body, 0)

        for k in range(_GRP):
            pltpu.sync_copy(
                gbufs[k].at[pl.ds(0, _NCELL * _ROWW)],
                out_hbm.at[pl.ds((gbase + k) * _NCELL * _ROWW,
                                 _NCELL * _ROWW)])
        return carry
    lax.fori_loop(0, _NGRP, group_body, 0)


def _mm_kernel(p_ref, we_ref, wo_ref, b_ref, o_ref):
    pw = lax.bitcast_convert_type(p_ref[:], jnp.int32)
    lowf = lax.bitcast_convert_type(pw << 16, jnp.float32)
    highf = lax.bitcast_convert_type(pw & jnp.int32(-65536), jnp.float32)
    dn = (((1,), (1,)), ((), ()))
    acc = lax.dot_general(we_ref[:], lowf, dn,
                          preferred_element_type=jnp.float32)
    acc = acc + lax.dot_general(wo_ref[:], highf, dn,
                                preferred_element_type=jnp.float32)
    o_ref[:] = acc + b_ref[:]


def kernel(hidden_states, positions, W, b):
    npd, hidd = hidden_states.shape
    px = positions[:, 0]
    py = positions[:, 1]
    hu = lax.bitcast_convert_type(
        hidden_states.astype(jnp.bfloat16).astype(jnp.float32), jnp.uint32)
    hword = (hu[:, :_ROWW] >> 16) | (hu[:, _ROWW:] & jnp.uint32(0xFFFF0000))
    hid_w = lax.bitcast_convert_type(hword, jnp.float32).reshape(-1)

    pool = pl.kernel(
        _sc_pool,
        out_type=jax.ShapeDtypeStruct((_NP * _NCELL * _ROWW,), jnp.float32),
        mesh=plsc.VectorSubcoreMesh(core_axis_name="c", subcore_axis_name="s"),
        compiler_params=pltpu.CompilerParams(needs_layout_passes=False),
        scratch_types=[
            pltpu.VMEM((_NP,), jnp.float32),
            pltpu.VMEM((_NP,), jnp.float32),
            pltpu.VMEM((_NP * _L,), jnp.int32),
            pltpu.VMEM((_NP * _ROWW,), jnp.float32),
        ] + [pltpu.VMEM(((_NCELL + 1) * _ROWW,), jnp.float32)
             for _ in range(_GRP)],
    )
    packed = pool(hid_w, px, py).reshape(npd, _NCELL * _ROWW)
    w3 = W.reshape(hidd, _NCELL, hidd)
    we = w3[:, :, :_ROWW].reshape(hidd, _NCELL * _ROWW)
    wo = w3[:, :, _ROWW:].reshape(hidd, _NCELL * _ROWW)

    rb = 128
    nw = _NCELL * _ROWW
    out = pl.pallas_call(
        _mm_kernel,
        grid=(npd // rb,),
        in_specs=[
            pl.BlockSpec((rb, nw), lambda i: (i, 0)),
            pl.BlockSpec((hidd, nw), lambda i: (0, 0)),
            pl.BlockSpec((hidd, nw), lambda i: (0, 0)),
            pl.BlockSpec((hidd, 1), lambda i: (0, 0)),
        ],
        out_specs=pl.BlockSpec((hidd, rb), lambda i: (0, i)),
        out_shape=jax.ShapeDtypeStruct((hidd, npd), jnp.float32),
    )(packed, we, wo, b.reshape(hidd, 1))
    return out.T


# speedup vs baseline: 37.3060x; 1.0032x over previous
import jax
import jax.numpy as jnp
from jax import lax
from jax.experimental import pallas as pl
from jax.experimental.pallas import tpu as pltpu
from jax.experimental.pallas import tpu_sc as plsc

_NS = 32.0
_G = 8
_NCELL = _G * _G
_HID = 128
_NP = 1024
_L = 16

_NWORK = 32
_CPW = _NP // _NWORK
_GRP = 8
_NGRP = _CPW // _GRP
_CHUNK = 256
_NCHUNK = _NP // _CHUNK
_ROWW = _HID // 2
_NT = _HID // (2 * _L)


def _lane_splat(v, lane):
    idx = jnp.full((_L,), lane, dtype=jnp.int32)
    dn = lax.GatherDimensionNumbers(
        offset_dims=(), collapsed_slice_dims=(0,), start_index_map=(0,))
    return lax.gather(v, idx[:, None], dn, (1,),
                      mode=lax.GatherScatterMode.PROMISE_IN_BOUNDS)


def _sc_pool(hid_hbm, px_hbm, py_hbm, out_hbm, pxv, pyv, cellr, hbuf,
             outsem, *gbufs):
    wid = lax.axis_index("s") * 2 + lax.axis_index("c")
    c0 = wid * _CPW
    pltpu.sync_copy(px_hbm, pxv)
    pltpu.sync_copy(py_hbm, pyv)
    pltpu.sync_copy(hid_hbm, hbuf)
    iota = lax.iota(jnp.int32, _L)
    half = _NS / 2
    inv = _G / _NS
    zero16 = jnp.zeros((_L,), jnp.float32)

    def group_body(g, carry):
        gbase = c0 + g * _GRP
        cxv = pxv[pl.ds(gbase, _L)]
        cyv = pyv[pl.ds(gbase, _L)]

        def zb(i, c):
            for k in range(_GRP):
                gbufs[k][pl.ds(i * _L, _L)] = zero16
            return c
        lax.fori_loop(0, (_NCELL + 1) * _ROWW // _L, zb, 0)

        for k in range(_GRP):
            cxk = _lane_splat(cxv, k)
            cyk = _lane_splat(cyv, k)
            cidk = gbase + k

            def cb(jg, c):
                jb = jg * _L
                jvec = jb + iota
                rx = pxv[pl.ds(jb, _L)] - cxk
                ry = pyv[pl.ds(jb, _L)] - cyk
                inb = (jnp.abs(rx) <= half) & (jnp.abs(ry) <= half)
                gx = jnp.clip((rx + half) * inv, 0.0, _G - 1.0)
                gy = jnp.clip((ry + half) * inv, 0.0, _G - 1.0)
                cell = gy.astype(jnp.int32) * _G + gx.astype(jnp.int32)
                ns = jvec != jnp.full((_L,), cidk, jnp.int32)
                row = jnp.where(inb & ns, cell, _NCELL) * _ROWW
                plsc.store_scatter(cellr, [jvec * _L + k], row)
                return c
            lax.fori_loop(0, _NP // _L, cb, 0)

        if True:
            def j_body(j, jc):
                rowv = cellr[pl.ds(j * _L, _L)]
                hv = [plsc.bitcast(hbuf[pl.ds(j * _ROWW + t * _L, _L)],
                                   jnp.bfloat16)
                      for t in range(_NT)]
                rks = [_lane_splat(rowv, k) for k in range(_GRP)]
                cvecs = [t * _L + iota for t in range(_NT)]
                gs = [None] * _NT
                ms = [None] * _NT
                for t in range(_NT):
                    gs[t] = [plsc.load_gather(gbufs[k], [rks[k] + cvecs[t]])
                             for k in range(_GRP)]
                    if t >= 1:
                        ms[t - 1] = [
                            plsc.bitcast(jnp.maximum(
                                plsc.bitcast(gs[t - 1][k], jnp.bfloat16),
                                hv[t - 1]), jnp.float32)
                            for k in range(_GRP)]
                        gs[t - 1] = None
                    if t >= 2:
                        for k in range(_GRP):
                            plsc.store_scatter(gbufs[k],
                                               [rks[k] + cvecs[t - 2]],
                                               ms[t - 2][k])
                        ms[t - 2] = None
                ms[_NT - 1] = [
                    plsc.bitcast(jnp.maximum(
                        plsc.bitcast(gs[_NT - 1][k], jnp.bfloat16),
                        hv[_NT - 1]), jnp.float32)
                    for k in range(_GRP)]
                for tt in (_NT - 2, _NT - 1):
                    for k in range(_GRP):
                        plsc.store_scatter(gbufs[k], [rks[k] + cvecs[tt]],
                                           ms[tt][k])
                return jc
            lax.fori_loop(0, _NP, j_body, 0)

        copies = [
            pltpu.async_copy(
                gbufs[k].at[pl.ds(0, _NCELL * _ROWW)],
                out_hbm.at[pl.ds((gbase + k) * _NCELL * _ROWW,
                                 _NCELL * _ROWW)],
                outsem)
            for k in range(_GRP)]
        for cp in copies:
            cp.wait()
        return carry
    lax.fori_loop(0, _NGRP, group_body, 0)


def _mm_kernel(p_ref, we_ref, wo_ref, b_ref, o_ref):
    pw = lax.bitcast_convert_type(p_ref[:], jnp.int32)
    lowf = lax.bitcast_convert_type(pw << 16, jnp.float32)
    highf = lax.bitcast_convert_type(pw & jnp.int32(-65536), jnp.float32)
    dn = (((1,), (1,)), ((), ()))
    acc = lax.dot_general(we_ref[:], lowf, dn,
                          preferred_element_type=jnp.float32)
    acc = acc + lax.dot_general(wo_ref[:], highf, dn,
                                preferred_element_type=jnp.float32)
    o_ref[:] = acc + b_ref[:]


def kernel(hidden_states, positions, W, b):
    npd, hidd = hidden_states.shape
    px = positions[:, 0]
    py = positions[:, 1]
    hu = lax.bitcast_convert_type(
        hidden_states.astype(jnp.bfloat16).astype(jnp.float32), jnp.uint32)
    hword = (hu[:, :_ROWW] >> 16) | (hu[:, _ROWW:] & jnp.uint32(0xFFFF0000))
    hid_w = lax.bitcast_convert_type(hword, jnp.float32).reshape(-1)

    pool = pl.kernel(
        _sc_pool,
        out_type=jax.ShapeDtypeStruct((_NP * _NCELL * _ROWW,), jnp.float32),
        mesh=plsc.VectorSubcoreMesh(core_axis_name="c", subcore_axis_name="s"),
        compiler_params=pltpu.CompilerParams(needs_layout_passes=False),
        scratch_types=[
            pltpu.VMEM((_NP,), jnp.float32),
            pltpu.VMEM((_NP,), jnp.float32),
            pltpu.VMEM((_NP * _L,), jnp.int32),
            pltpu.VMEM((_NP * _ROWW,), jnp.float32),
            pltpu.SemaphoreType.DMA,
        ] + [pltpu.VMEM(((_NCELL + 1) * _ROWW,), jnp.float32)
             for _ in range(_GRP)],
    )
    packed = pool(hid_w, px, py).reshape(npd, _NCELL * _ROWW)
    w3 = W.reshape(hidd, _NCELL, hidd)
    we = w3[:, :, :_ROWW].reshape(hidd, _NCELL * _ROWW)
    wo = w3[:, :, _ROWW:].reshape(hidd, _NCELL * _ROWW)

    rb = 128
    nw = _NCELL * _ROWW
    out = pl.pallas_call(
        _mm_kernel,
        grid=(npd // rb,),
        in_specs=[
            pl.BlockSpec((rb, nw), lambda i: (i, 0)),
            pl.BlockSpec((hidd, nw), lambda i: (0, 0)),
            pl.BlockSpec((hidd, nw), lambda i: (0, 0)),
            pl.BlockSpec((hidd, 1), lambda i: (0, 0)),
        ],
        out_specs=pl.BlockSpec((hidd, rb), lambda i: (0, i)),
        out_shape=jax.ShapeDtypeStruct((hidd, npd), jnp.float32),
    )(packed, we, wo, b.reshape(hidd, 1))
    return out.T


# speedup vs baseline: 39.1700x; 1.0500x over previous
import jax
import jax.numpy as jnp
from jax import lax
from jax.experimental import pallas as pl
from jax.experimental.pallas import tpu as pltpu
from jax.experimental.pallas import tpu_sc as plsc

_NS = 32.0
_G = 8
_NCELL = _G * _G
_HID = 128
_NP = 1024
_L = 16

_NWORK = 32
_CPW = _NP // _NWORK
_GRP = 8
_NGRP = _CPW // _GRP
_CHUNK = 256
_NCHUNK = _NP // _CHUNK
_ROWW = _HID // 2
_NT = _HID // (2 * _L)


def _lane_splat(v, lane):
    idx = jnp.full((_L,), lane, dtype=jnp.int32)
    dn = lax.GatherDimensionNumbers(
        offset_dims=(), collapsed_slice_dims=(0,), start_index_map=(0,))
    return lax.gather(v, idx[:, None], dn, (1,),
                      mode=lax.GatherScatterMode.PROMISE_IN_BOUNDS)


def _sc_pool(hid_hbm, px_hbm, py_hbm, out_hbm, pxv, pyv, cellr, hbuf,
             outsem, *gbufs):
    wid = lax.axis_index("s") * 2 + lax.axis_index("c")
    c0 = wid * _CPW
    pltpu.sync_copy(px_hbm, pxv)
    pltpu.sync_copy(py_hbm, pyv)
    pltpu.sync_copy(hid_hbm, hbuf)
    iota = lax.iota(jnp.int32, _L)
    half = _NS / 2
    inv = _G / _NS
    zero16 = jnp.zeros((_L,), jnp.float32)

    def group_body(g, carry):
        gbase = c0 + g * _GRP
        cxv = pxv[pl.ds(gbase, _L)]
        cyv = pyv[pl.ds(gbase, _L)]

        def zb(i, c):
            for k in range(_GRP):
                gbufs[k][pl.ds(i * _L, _L)] = zero16
            return c
        lax.fori_loop(0, (_NCELL + 1) * _ROWW // _L, zb, 0)

        cxks = [_lane_splat(cxv, k) for k in range(_GRP)]
        cyks = [_lane_splat(cyv, k) for k in range(_GRP)]

        def cb(jg, c):
            jb = jg * _L
            jvec = jb + iota
            pxj = pxv[pl.ds(jb, _L)]
            pyj = pyv[pl.ds(jb, _L)]
            for k in range(_GRP):
                rx = pxj - cxks[k]
                ry = pyj - cyks[k]
                inb = (jnp.abs(rx) <= half) & (jnp.abs(ry) <= half)
                gx = jnp.clip((rx + half) * inv, 0.0, _G - 1.0)
                gy = jnp.clip((ry + half) * inv, 0.0, _G - 1.0)
                cell = gy.astype(jnp.int32) * _G + gx.astype(jnp.int32)
                ns = jvec != jnp.full((_L,), gbase + k, jnp.int32)
                row = jnp.where(inb & ns, cell, _NCELL) * _ROWW
                plsc.store_scatter(cellr, [jvec * _L + k], row)
            return c
        lax.fori_loop(0, _NP // _L, cb, 0)

        if True:
            def j_body(j, jc):
                rowv = cellr[pl.ds(j * _L, _L)]
                hv = [plsc.bitcast(hbuf[pl.ds(j * _ROWW + t * _L, _L)],
                                   jnp.bfloat16)
                      for t in range(_NT)]
                rks = [_lane_splat(rowv, k) for k in range(_GRP)]
                cvecs = [t * _L + iota for t in range(_NT)]
                gs = [None] * _NT
                ms = [None] * _NT
                for t in range(_NT):
                    gs[t] = [plsc.load_gather(gbufs[k], [rks[k] + cvecs[t]])
                             for k in range(_GRP)]
                    if t >= 1:
                        ms[t - 1] = [
                            plsc.bitcast(jnp.maximum(
                                plsc.bitcast(gs[t - 1][k], jnp.bfloat16),
                                hv[t - 1]), jnp.float32)
                            for k in range(_GRP)]
                        gs[t - 1] = None
                    if t >= 2:
                        for k in range(_GRP):
                            plsc.store_scatter(gbufs[k],
                                               [rks[k] + cvecs[t - 2]],
                                               ms[t - 2][k])
                        ms[t - 2] = None
                ms[_NT - 1] = [
                    plsc.bitcast(jnp.maximum(
                        plsc.bitcast(gs[_NT - 1][k], jnp.bfloat16),
                        hv[_NT - 1]), jnp.float32)
                    for k in range(_GRP)]
                for tt in (_NT - 2, _NT - 1):
                    for k in range(_GRP):
                        plsc.store_scatter(gbufs[k], [rks[k] + cvecs[tt]],
                                           ms[tt][k])
                return jc
            lax.fori_loop(0, _NP, j_body, 0)

        copies = [
            pltpu.async_copy(
                gbufs[k].at[pl.ds(0, _NCELL * _ROWW)],
                out_hbm.at[pl.ds((gbase + k) * _NCELL * _ROWW,
                                 _NCELL * _ROWW)],
                outsem)
            for k in range(_GRP)]
        for cp in copies:
            cp.wait()
        return carry
    lax.fori_loop(0, _NGRP, group_body, 0)


def _mm_kernel(p_ref, we_ref, wo_ref, b_ref, o_ref):
    pw = lax.bitcast_convert_type(p_ref[:], jnp.int32)
    lowf = lax.bitcast_convert_type(pw << 16, jnp.float32)
    highf = lax.bitcast_convert_type(pw & jnp.int32(-65536), jnp.float32)
    dn = (((1,), (1,)), ((), ()))
    acc = lax.dot_general(we_ref[:], lowf, dn,
                          preferred_element_type=jnp.float32)
    acc = acc + lax.dot_general(wo_ref[:], highf, dn,
                                preferred_element_type=jnp.float32)
    o_ref[:] = acc + b_ref[:]


def kernel(hidden_states, positions, W, b):
    npd, hidd = hidden_states.shape
    px = positions[:, 0]
    py = positions[:, 1]
    hu = lax.bitcast_convert_type(
        hidden_states.astype(jnp.bfloat16).astype(jnp.float32), jnp.uint32)
    hword = (hu[:, :_ROWW] >> 16) | (hu[:, _ROWW:] & jnp.uint32(0xFFFF0000))
    hid_w = lax.bitcast_convert_type(hword, jnp.float32).reshape(-1)

    pool = pl.kernel(
        _sc_pool,
        out_type=jax.ShapeDtypeStruct((_NP * _NCELL * _ROWW,), jnp.float32),
        mesh=plsc.VectorSubcoreMesh(core_axis_name="c", subcore_axis_name="s"),
        compiler_params=pltpu.CompilerParams(needs_layout_passes=False),
        scratch_types=[
            pltpu.VMEM((_NP,), jnp.float32),
            pltpu.VMEM((_NP,), jnp.float32),
            pltpu.VMEM((_NP * _L,), jnp.int32),
            pltpu.VMEM((_NP * _ROWW,), jnp.float32),
            pltpu.SemaphoreType.DMA,
        ] + [pltpu.VMEM(((_NCELL + 1) * _ROWW,), jnp.float32)
             for _ in range(_GRP)],
    )
    packed = pool(hid_w, px, py).reshape(npd, _NCELL * _ROWW)
    w3 = W.reshape(hidd, _NCELL, hidd)
    we = w3[:, :, :_ROWW].reshape(hidd, _NCELL * _ROWW)
    wo = w3[:, :, _ROWW:].reshape(hidd, _NCELL * _ROWW)

    rb = 128
    nw = _NCELL * _ROWW
    out = pl.pallas_call(
        _mm_kernel,
        grid=(npd // rb,),
        in_specs=[
            pl.BlockSpec((rb, nw), lambda i: (i, 0)),
            pl.BlockSpec((hidd, nw), lambda i: (0, 0)),
            pl.BlockSpec((hidd, nw), lambda i: (0, 0)),
            pl.BlockSpec((hidd, 1), lambda i: (0, 0)),
        ],
        out_specs=pl.BlockSpec((hidd, rb), lambda i: (0, i)),
        out_shape=jax.ShapeDtypeStruct((hidd, npd), jnp.float32),
    )(packed, we, wo, b.reshape(hidd, 1))
    return out.T
